# Initial kernel scaffold; baseline (speedup 1.0000x reference)
import jax
import jax.numpy as jnp
from jax.experimental import pallas as pl


def kernel(x, edge_index, edge_attr, W_in, b_in, W_e, b_e, W_h, b_h):
    raise NotImplementedError("write your pallas kernel here")


# speedup vs baseline: 2.3657x; 2.3657x over previous
import functools

import jax
import jax.numpy as jnp
from jax import lax
from jax.experimental import pallas as pl
from jax.experimental.pallas import tpu as pltpu
from jax.experimental.pallas import tpu_sc as plsc

_NC = 2
_NS = 16
_CH = 128


def _tc_proj(x, W, b, blk):
    M, K = x.shape
    Do = W.shape[1]

    def body(x_ref, w_ref, b_ref, o_ref):
        v = jnp.dot(x_ref[...], w_ref[...], preferred_element_type=jnp.float32)
        o_ref[...] = jnp.maximum(v + b_ref[...], 0.0)

    return pl.pallas_call(
        body,
        grid=(M // blk,),
        in_specs=[
            pl.BlockSpec((blk, K), lambda i: (i, 0)),
            pl.BlockSpec((K, Do), lambda i: (0, 0)),
            pl.BlockSpec((1, Do), lambda i: (0, 0)),
        ],
        out_specs=pl.BlockSpec((blk, Do), lambda i: (i, 0)),
        out_shape=jax.ShapeDtypeStruct((M, Do), jnp.float32),
    )(x, W, b.reshape(1, Do))


def _tc_edge_proj(ea_pad, W, b, n_real, blk):
    M, K = ea_pad.shape
    Do = W.shape[1]

    def body(a_ref, w_ref, b_ref, o_ref):
        i = pl.program_id(0)
        v = jnp.dot(a_ref[...], w_ref[...], preferred_element_type=jnp.float32)
        v = jnp.maximum(v + b_ref[...], 0.0)
        rows = i * blk + lax.broadcasted_iota(jnp.int32, v.shape, 0)
        o_ref[...] = jnp.where(rows < n_real, v, jnp.float32(-1e30))

    return pl.pallas_call(
        body,
        grid=(M // blk,),
        in_specs=[
            pl.BlockSpec((blk, K), lambda i: (i, 0)),
            pl.BlockSpec((K, Do), lambda i: (0, 0)),
            pl.BlockSpec((1, Do), lambda i: (0, 0)),
        ],
        out_specs=pl.BlockSpec((blk, Do), lambda i: (i, 0)),
        out_shape=jax.ShapeDtypeStruct((M, Do), jnp.float32),
    )(ea_pad, W, b.reshape(1, Do))


def _tc_update(h, a0, a1, W, b, blk):
    M, K = h.shape
    Do = W.shape[1]

    def body(h_ref, a0_ref, a1_ref, w_ref, b_ref, o_ref):
        t = h_ref[...] + a0_ref[...] + a1_ref[...]
        v = jnp.dot(t, w_ref[...], preferred_element_type=jnp.float32)
        o_ref[...] = jnp.maximum(v + b_ref[...], 0.0)

    return pl.pallas_call(
        body,
        grid=(M // blk,),
        in_specs=[
            pl.BlockSpec((blk, K), lambda i: (i, 0)),
            pl.BlockSpec((blk, K), lambda i: (i, 0)),
            pl.BlockSpec((blk, K), lambda i: (i, 0)),
            pl.BlockSpec((K, Do), lambda i: (0, 0)),
            pl.BlockSpec((1, Do), lambda i: (0, 0)),
        ],
        out_specs=pl.BlockSpec((blk, Do), lambda i: (i, 0)),
        out_shape=jax.ShapeDtypeStruct((M, Do), jnp.float32),
    )(h, a0, a1, W, b.reshape(1, Do))


@functools.cache
def _make_sc_layer(N, D, E_pad, N_pad):
    NW = _NC * _NS
    EPW = E_pad // NW
    NCHK = EPW // _CH
    RPT = N_pad // _NS
    mesh = plsc.VectorSubcoreMesh(core_axis_name="c", subcore_axis_name="s")

    @functools.partial(
        pl.kernel,
        out_type=jax.ShapeDtypeStruct((_NC * N_pad, D), jnp.float32),
        mesh=mesh,
        scratch_types=[
            pltpu.VMEM((_CH,), jnp.int32),
            pltpu.VMEM((_CH,), jnp.int32),
            pltpu.VMEM((_CH, D), jnp.float32),
            pltpu.VMEM((_CH, D), jnp.float32),
            pltpu.VMEM_SHARED((N_pad, D), jnp.float32),
            pltpu.SemaphoreType.DMA,
        ],
    )
    def sc_layer(h_hbm, e_hbm, src_hbm, dst_hbm, out_hbm,
                 srcv, dstv, hbuf, ebuf, agg_sh, sem):
        c = lax.axis_index("c")
        s = lax.axis_index("s")
        wid = c * _NS + s

        def zrow(r, carry):
            for j in range(D // 16):
                hbuf[r, pl.ds(j * 16, 16)] = jnp.zeros((16,), jnp.float32)
            return carry

        lax.fori_loop(0, _CH, zrow, 0)
        for t in range(RPT // _CH):
            pltpu.sync_copy(hbuf, agg_sh.at[pl.ds(s * RPT + t * _CH, _CH)])
        plsc.subcore_barrier()

        def chunk(g, carry):
            base = wid * EPW + g * _CH
            pltpu.sync_copy(src_hbm.at[pl.ds(base, _CH)], srcv)
            pltpu.sync_copy(dst_hbm.at[pl.ds(base, _CH)], dstv)
            pltpu.sync_copy(e_hbm.at[pl.ds(base, _CH)], ebuf)
            pltpu.async_copy(h_hbm.at[srcv], hbuf, sem).wait()

            def row(r, rc):
                for j in range(D // 16):
                    sl = pl.ds(j * 16, 16)
                    hbuf[r, sl] = jnp.maximum(hbuf[r, sl] + ebuf[r, sl], 0.0)
                return rc

            lax.fori_loop(0, _CH, row, 0)
            pltpu.sync_copy(hbuf, agg_sh.at[dstv], add=True)
            return carry

        lax.fori_loop(0, NCHK, chunk, 0)
        plsc.subcore_barrier()
        pltpu.sync_copy(agg_sh.at[pl.ds(s * RPT, RPT)],
                        out_hbm.at[pl.ds(c * N_pad + s * RPT, RPT)])

    return sc_layer


def kernel(x, edge_index, edge_attr, W_in, b_in, W_e, b_e, W_h, b_h):
    N, D = x.shape
    E = edge_index.shape[1]
    depth = W_h.shape[0]

    grain = _NC * _NS * _CH
    E_pad = ((E + grain - 1) // grain) * grain
    pad = E_pad - E
    N_pad = ((N + _NS * _CH - 1) // (_NS * _CH)) * (_NS * _CH)

    src = jnp.concatenate([edge_index[0].astype(jnp.int32),
                           jnp.zeros((pad,), jnp.int32)])
    dst = jnp.concatenate([edge_index[1].astype(jnp.int32),
                           jnp.zeros((pad,), jnp.int32)])
    ea_pad = jnp.pad(edge_attr, ((0, pad), (0, 0)))

    h = _tc_proj(x, W_in, b_in, blk=2000)
    e = _tc_edge_proj(ea_pad, W_e, b_e, n_real=E, blk=2048)

    sc_layer = _make_sc_layer(N, D, E_pad, N_pad)
    for i in range(depth):
        agg = sc_layer(h, e, src, dst)
        h = _tc_update(h, agg[:N], agg[N_pad:N_pad + N], W_h[i], b_h[i],
                       blk=2000)
    return h


# speedup vs baseline: 2.6556x; 1.1225x over previous
import functools

import jax
import jax.numpy as jnp
from jax import lax
from jax.experimental import pallas as pl
from jax.experimental.pallas import tpu as pltpu
from jax.experimental.pallas import tpu_sc as plsc

_NC = 2
_NS = 16
_CH = 64


def _tc_proj(x, W, b, blk):
    M, K = x.shape
    Do = W.shape[1]

    def body(x_ref, w_ref, b_ref, o_ref):
        v = jnp.dot(x_ref[...], w_ref[...], preferred_element_type=jnp.float32)
        o_ref[...] = jnp.maximum(v + b_ref[...], 0.0)

    return pl.pallas_call(
        body,
        grid=(M // blk,),
        in_specs=[
            pl.BlockSpec((blk, K), lambda i: (i, 0)),
            pl.BlockSpec((K, Do), lambda i: (0, 0)),
            pl.BlockSpec((1, Do), lambda i: (0, 0)),
        ],
        out_specs=pl.BlockSpec((blk, Do), lambda i: (i, 0)),
        out_shape=jax.ShapeDtypeStruct((M, Do), jnp.float32),
    )(x, W, b.reshape(1, Do))


def _tc_edge_proj(ea_pad, W, b, n_real, blk):
    M, K = ea_pad.shape
    Do = W.shape[1]

    def body(a_ref, w_ref, b_ref, o_ref):
        i = pl.program_id(0)
        v = jnp.dot(a_ref[...], w_ref[...], preferred_element_type=jnp.float32)
        v = jnp.maximum(v + b_ref[...], 0.0)
        rows = i * blk + lax.broadcasted_iota(jnp.int32, v.shape, 0)
        o_ref[...] = jnp.where(rows < n_real, v, jnp.float32(-1e30))

    return pl.pallas_call(
        body,
        grid=(M // blk,),
        in_specs=[
            pl.BlockSpec((blk, K), lambda i: (i, 0)),
            pl.BlockSpec((K, Do), lambda i: (0, 0)),
            pl.BlockSpec((1, Do), lambda i: (0, 0)),
        ],
        out_specs=pl.BlockSpec((blk, Do), lambda i: (i, 0)),
        out_shape=jax.ShapeDtypeStruct((M, Do), jnp.float32),
    )(ea_pad, W, b.reshape(1, Do))


def _tc_update(h, a0, a1, W, b, blk):
    M, K = h.shape
    Do = W.shape[1]

    def body(h_ref, a0_ref, a1_ref, w_ref, b_ref, o_ref):
        t = h_ref[...] + a0_ref[...] + a1_ref[...]
        v = jnp.dot(t, w_ref[...], preferred_element_type=jnp.float32)
        o_ref[...] = jnp.maximum(v + b_ref[...], 0.0)

    return pl.pallas_call(
        body,
        grid=(M // blk,),
        in_specs=[
            pl.BlockSpec((blk, K), lambda i: (i, 0)),
            pl.BlockSpec((blk, K), lambda i: (i, 0)),
            pl.BlockSpec((blk, K), lambda i: (i, 0)),
            pl.BlockSpec((K, Do), lambda i: (0, 0)),
            pl.BlockSpec((1, Do), lambda i: (0, 0)),
        ],
        out_specs=pl.BlockSpec((blk, Do), lambda i: (i, 0)),
        out_shape=jax.ShapeDtypeStruct((M, Do), jnp.float32),
    )(h, a0, a1, W, b.reshape(1, Do))


@functools.cache
def _make_sc_layer(N, D, E_pad, N_pad):
    NW = _NC * _NS
    NCHK = E_pad // (NW * _CH)
    RPT = N_pad // _NS
    ZR = (RPT + _CH - 1) // _CH
    mesh = plsc.VectorSubcoreMesh(core_axis_name="c", subcore_axis_name="s")

    @functools.partial(
        pl.kernel,
        out_type=jax.ShapeDtypeStruct((_NC * N_pad, D), jnp.float32),
        mesh=mesh,
        scratch_types=[
            pltpu.VMEM((4, _CH), jnp.int32),
            pltpu.VMEM((4, _CH), jnp.int32),
            pltpu.VMEM((2, _CH, D), jnp.float32),
            pltpu.VMEM((2, _CH, D), jnp.float32),
            pltpu.VMEM((2, _CH, D), jnp.float32),
            pltpu.VMEM_SHARED((N_pad, D), jnp.float32),
            pltpu.SemaphoreType.DMA,
            pltpu.SemaphoreType.DMA,
            pltpu.SemaphoreType.DMA,
            pltpu.SemaphoreType.DMA,
            pltpu.SemaphoreType.DMA,
            pltpu.SemaphoreType.DMA,
            pltpu.SemaphoreType.DMA,
            pltpu.SemaphoreType.DMA,
            pltpu.SemaphoreType.DMA,
            pltpu.SemaphoreType.DMA,
        ],
    )
    def sc_layer(h_hbm, e_hbm, src_hbm, dst_hbm, out_hbm,
                 srcv, dstv, hbuf, ebuf, mbuf, agg_sh,
                 gsem0, gsem1, esem0, esem1, ssem0, ssem1,
                 isem0, isem1, dsem0, dsem1):
        gsem = (gsem0, gsem1)
        esem = (esem0, esem1)
        ssem = (ssem0, ssem1)
        isem = (isem0, isem1)
        dsem = (dsem0, dsem1)
        c = lax.axis_index("c")
        s = lax.axis_index("s")
        wid = c * _NS + s
        tb = wid * NCHK

        def wait_e(p):
            pltpu.make_async_copy(e_hbm.at[pl.ds(0, _CH)], ebuf.at[p],
                                  esem[p]).wait()

        def wait_g(p):
            pltpu.make_async_copy(h_hbm.at[srcv.at[0]], hbuf.at[p],
                                  gsem[p]).wait()

        def wait_s(p):
            pltpu.make_async_copy(mbuf.at[p], agg_sh.at[dstv.at[0]],
                                  ssem[p]).wait()

        def wait_idx(ring, sem):
            pltpu.make_async_copy(src_hbm.at[pl.ds(0, _CH)], ring.at[0],
                                  sem).wait()

        def zrow(r, carry):
            for j in range(D // 16):
                mbuf[0, r, pl.ds(j * 16, 16)] = jnp.zeros((16,), jnp.float32)
            return carry

        lax.fori_loop(0, _CH, zrow, 0)
        for t in range(ZR):
            rows = min(_CH, RPT - t * _CH)
            pltpu.sync_copy(mbuf.at[0, pl.ds(0, rows)],
                            agg_sh.at[pl.ds(s * RPT + t * _CH, rows)])

        for k in range(4):
            pltpu.sync_copy(src_hbm.at[pl.ds((tb + k) * _CH, _CH)],
                            srcv.at[k])
        for k in range(2):
            pltpu.sync_copy(dst_hbm.at[pl.ds((tb + k) * _CH, _CH)],
                            dstv.at[k])
        for k in range(2):
            pltpu.async_copy(e_hbm.at[pl.ds((tb + k) * _CH, _CH)],
                             ebuf.at[k], esem[k])
            pltpu.async_copy(h_hbm.at[srcv.at[k]], hbuf.at[k], gsem[k])
        plsc.subcore_barrier()

        def quad(g, carry):
            for u in range(4):
                k = g * 4 + u
                p = u % 2
                wait_e(p)
                wait_g(p)

                @pl.when(k >= 2)
                def _():
                    wait_s(p)

                @pl.when(k + 2 < NCHK)
                def _():
                    pltpu.async_copy(
                        dst_hbm.at[pl.ds((tb + k + 2) * _CH, _CH)],
                        dstv.at[(u + 2) % 4], dsem[[1, 1, 0, 0][u]])

                @pl.when(k + 4 < NCHK)
                def _():
                    pltpu.async_copy(
                        src_hbm.at[pl.ds((tb + k + 4) * _CH, _CH)],
                        srcv.at[u], isem[[0, 0, 1, 1][u]])

                def row(r, rc):
                    for rr in range(2):
                        for j in range(D // 16):
                            sl = pl.ds(j * 16, 16)
                            mbuf[p, r * 2 + rr, sl] = jnp.maximum(
                                hbuf[p, r * 2 + rr, sl]
                                + ebuf[p, r * 2 + rr, sl], 0.0)
                    return rc

                lax.fori_loop(0, _CH // 2, row, 0)

                @pl.when(k >= 2)
                def _():
                    wait_idx(dstv, dsem[[0, 0, 1, 1][u]])

                pltpu.async_copy(mbuf.at[p], agg_sh.at[dstv.at[u]],
                                 ssem[p], add=True)

                @pl.when(jnp.logical_and(k >= 2, k + 2 < NCHK))
                def _():
                    wait_idx(srcv, isem[[1, 1, 0, 0][u]])

                @pl.when(k + 2 < NCHK)
                def _():
                    pltpu.async_copy(
                        e_hbm.at[pl.ds((tb + k + 2) * _CH, _CH)],
                        ebuf.at[p], esem[p])
                    pltpu.async_copy(h_hbm.at[srcv.at[(u + 2) % 4]],
                                     hbuf.at[p], gsem[p])
            return carry

        lax.fori_loop(0, NCHK // 4, quad, 0)
        for p in range(2):
            wait_s(p)
        plsc.subcore_barrier()
        pltpu.sync_copy(agg_sh.at[pl.ds(s * RPT, RPT)],
                        out_hbm.at[pl.ds(c * N_pad + s * RPT, RPT)])

    return sc_layer


def kernel(x, edge_index, edge_attr, W_in, b_in, W_e, b_e, W_h, b_h):
    N, D = x.shape
    E = edge_index.shape[1]
    depth = W_h.shape[0]

    grain = _NC * _NS * _CH * 4
    E_pad = ((E + grain - 1) // grain) * grain
    pad = E_pad - E
    N_pad = ((N + _NS * 8 - 1) // (_NS * 8)) * (_NS * 8)

    src = jnp.concatenate([edge_index[0].astype(jnp.int32),
                           jnp.zeros((pad,), jnp.int32)])
    dst = jnp.concatenate([edge_index[1].astype(jnp.int32),
                           jnp.zeros((pad,), jnp.int32)])
    ea_pad = jnp.pad(edge_attr, ((0, pad), (0, 0)))

    h = _tc_proj(x, W_in, b_in, blk=2000)
    e = _tc_edge_proj(ea_pad, W_e, b_e, n_real=E, blk=2048)

    sc_layer = _make_sc_layer(N, D, E_pad, N_pad)
    for i in range(depth):
        agg = sc_layer(h, e, src, dst)
        h = _tc_update(h, agg[:N], agg[N_pad:N_pad + N], W_h[i], b_h[i],
                       blk=2000)
    return h


# speedup vs baseline: 2.8412x; 1.0699x over previous
import functools

import jax
import jax.numpy as jnp
from jax import lax
from jax.experimental import pallas as pl
from jax.experimental.pallas import tpu as pltpu
from jax.experimental.pallas import tpu_sc as plsc

_NC = 2
_NS = 16
_CH = 64


def _tc_proj(x, W, b, blk):
    M, K = x.shape
    Do = W.shape[1]

    def body(x_ref, w_ref, b_ref, o_ref):
        v = jnp.dot(x_ref[...], w_ref[...], preferred_element_type=jnp.float32)
        o_ref[...] = jnp.maximum(v + b_ref[...], 0.0)

    return pl.pallas_call(
        body,
        grid=(M // blk,),
        in_specs=[
            pl.BlockSpec((blk, K), lambda i: (i, 0)),
            pl.BlockSpec((K, Do), lambda i: (0, 0)),
            pl.BlockSpec((1, Do), lambda i: (0, 0)),
        ],
        out_specs=pl.BlockSpec((blk, Do), lambda i: (i, 0)),
        out_shape=jax.ShapeDtypeStruct((M, Do), jnp.float32),
    )(x, W, b.reshape(1, Do))


def _tc_edge_proj(ea_pad, W, b, n_real, blk):
    M, K = ea_pad.shape
    Do = W.shape[1]

    def body(a_ref, w_ref, b_ref, o_ref):
        i = pl.program_id(0)
        v = jnp.dot(a_ref[...], w_ref[...], preferred_element_type=jnp.float32)
        v = jnp.maximum(v + b_ref[...], 0.0)
        rows = i * blk + lax.broadcasted_iota(jnp.int32, v.shape, 0)
        o_ref[...] = jnp.where(rows < n_real, v, jnp.float32(-1e30))

    return pl.pallas_call(
        body,
        grid=(M // blk,),
        in_specs=[
            pl.BlockSpec((blk, K), lambda i: (i, 0)),
            pl.BlockSpec((K, Do), lambda i: (0, 0)),
            pl.BlockSpec((1, Do), lambda i: (0, 0)),
        ],
        out_specs=pl.BlockSpec((blk, Do), lambda i: (i, 0)),
        out_shape=jax.ShapeDtypeStruct((M, Do), jnp.float32),
    )(ea_pad, W, b.reshape(1, Do))


def _tc_update(h, a0, a1, W, b, blk):
    M, K = h.shape
    Do = W.shape[1]

    def body(h_ref, a0_ref, a1_ref, w_ref, b_ref, o_ref):
        t = h_ref[...] + a0_ref[...] + a1_ref[...]
        v = jnp.dot(t, w_ref[...], preferred_element_type=jnp.float32)
        o_ref[...] = jnp.maximum(v + b_ref[...], 0.0)

    return pl.pallas_call(
        body,
        grid=(M // blk,),
        in_specs=[
            pl.BlockSpec((blk, K), lambda i: (i, 0)),
            pl.BlockSpec((blk, K), lambda i: (i, 0)),
            pl.BlockSpec((blk, K), lambda i: (i, 0)),
            pl.BlockSpec((K, Do), lambda i: (0, 0)),
            pl.BlockSpec((1, Do), lambda i: (0, 0)),
        ],
        out_specs=pl.BlockSpec((blk, Do), lambda i: (i, 0)),
        out_shape=jax.ShapeDtypeStruct((M, Do), jnp.float32),
    )(h, a0, a1, W, b.reshape(1, Do))


@functools.cache
def _make_sc_layer(N, D, E_pad, N_pad, C0, C1):
    assert (C0 + C1) * _NS * _CH == E_pad and C0 % 4 == 0 and C1 % 4 == 0
    RPT = N_pad // _NS
    ZR = (RPT + _CH - 1) // _CH
    mesh = plsc.VectorSubcoreMesh(core_axis_name="c", subcore_axis_name="s")

    @functools.partial(
        pl.kernel,
        out_type=jax.ShapeDtypeStruct((_NC * N_pad, D), jnp.float32),
        mesh=mesh,
        scratch_types=[
            pltpu.VMEM((4, _CH), jnp.int32),
            pltpu.VMEM((4, _CH), jnp.int32),
            pltpu.VMEM((2, _CH, D), jnp.float32),
            pltpu.VMEM((2, _CH, D), jnp.float32),
            pltpu.VMEM((2, _CH, D), jnp.float32),
            pltpu.VMEM_SHARED((N_pad, D), jnp.float32),
            pltpu.SemaphoreType.DMA,
            pltpu.SemaphoreType.DMA,
            pltpu.SemaphoreType.DMA,
            pltpu.SemaphoreType.DMA,
            pltpu.SemaphoreType.DMA,
            pltpu.SemaphoreType.DMA,
            pltpu.SemaphoreType.DMA,
            pltpu.SemaphoreType.DMA,
            pltpu.SemaphoreType.DMA,
            pltpu.SemaphoreType.DMA,
        ],
    )
    def sc_layer(h_hbm, e_hbm, src_hbm, dst_hbm, out_hbm,
                 srcv, dstv, hbuf, ebuf, mbuf, agg_sh,
                 gsem0, gsem1, esem0, esem1, ssem0, ssem1,
                 isem0, isem1, dsem0, dsem1):
        gsem = (gsem0, gsem1)
        esem = (esem0, esem1)
        ssem = (ssem0, ssem1)
        isem = (isem0, isem1)
        dsem = (dsem0, dsem1)
        c = lax.axis_index("c")
        s = lax.axis_index("s")
        nchk = jnp.where(c == 0, C0, C1)
        tb = jnp.where(c == 0, s * C0, _NS * C0 + s * C1)

        def wait_e(p):
            pltpu.make_async_copy(e_hbm.at[pl.ds(0, _CH)], ebuf.at[p],
                                  esem[p]).wait()

        def wait_g(p):
            pltpu.make_async_copy(h_hbm.at[srcv.at[0]], hbuf.at[p],
                                  gsem[p]).wait()

        def wait_s(p):
            pltpu.make_async_copy(mbuf.at[p], agg_sh.at[dstv.at[0]],
                                  ssem[p]).wait()

        def wait_idx(ring, sem):
            pltpu.make_async_copy(src_hbm.at[pl.ds(0, _CH)], ring.at[0],
                                  sem).wait()

        def zrow(r, carry):
            for j in range(D // 16):
                mbuf[0, r, pl.ds(j * 16, 16)] = jnp.zeros((16,), jnp.float32)
            return carry

        lax.fori_loop(0, _CH, zrow, 0)
        for t in range(ZR):
            rows = min(_CH, RPT - t * _CH)
            pltpu.sync_copy(mbuf.at[0, pl.ds(0, rows)],
                            agg_sh.at[pl.ds(s * RPT + t * _CH, rows)])

        for k in range(4):
            pltpu.sync_copy(src_hbm.at[pl.ds((tb + k) * _CH, _CH)],
                            srcv.at[k])
        for k in range(2):
            pltpu.sync_copy(dst_hbm.at[pl.ds((tb + k) * _CH, _CH)],
                            dstv.at[k])
        for k in range(2):
            pltpu.async_copy(e_hbm.at[pl.ds((tb + k) * _CH, _CH)],
                             ebuf.at[k], esem[k])
            pltpu.async_copy(h_hbm.at[srcv.at[k]], hbuf.at[k], gsem[k])
        plsc.subcore_barrier()

        def quad(g, carry):
            for u in range(4):
                k = g * 4 + u
                p = u % 2
                wait_e(p)
                wait_g(p)

                @pl.when(k >= 2)
                def _():
                    wait_s(p)

                @pl.when(k + 2 < nchk)
                def _():
                    pltpu.async_copy(
                        dst_hbm.at[pl.ds((tb + k + 2) * _CH, _CH)],
                        dstv.at[(u + 2) % 4], dsem[[1, 1, 0, 0][u]])

                @pl.when(k + 4 < nchk)
                def _():
                    pltpu.async_copy(
                        src_hbm.at[pl.ds((tb + k + 4) * _CH, _CH)],
                        srcv.at[u], isem[[0, 0, 1, 1][u]])

                def row(r, rc):
                    for rr in range(2):
                        for j in range(D // 16):
                            sl = pl.ds(j * 16, 16)
                            mbuf[p, r * 2 + rr, sl] = jnp.maximum(
                                hbuf[p, r * 2 + rr, sl]
                                + ebuf[p, r * 2 + rr, sl], 0.0)
                    return rc

                lax.fori_loop(0, _CH // 2, row, 0)

                @pl.when(k >= 2)
                def _():
                    wait_idx(dstv, dsem[[0, 0, 1, 1][u]])

                pltpu.async_copy(mbuf.at[p], agg_sh.at[dstv.at[u]],
                                 ssem[p], add=True)

                @pl.when(jnp.logical_and(k >= 2, k + 2 < nchk))
                def _():
                    wait_idx(srcv, isem[[1, 1, 0, 0][u]])

                @pl.when(k + 2 < nchk)
                def _():
                    pltpu.async_copy(
                        e_hbm.at[pl.ds((tb + k + 2) * _CH, _CH)],
                        ebuf.at[p], esem[p])
                    pltpu.async_copy(h_hbm.at[srcv.at[(u + 2) % 4]],
                                     hbuf.at[p], gsem[p])
            return carry

        lax.fori_loop(0, nchk // 4, quad, 0)
        for p in range(2):
            wait_s(p)
        plsc.subcore_barrier()
        pltpu.sync_copy(agg_sh.at[pl.ds(s * RPT, RPT)],
                        out_hbm.at[pl.ds(c * N_pad + s * RPT, RPT)])

    return sc_layer


def kernel(x, edge_index, edge_attr, W_in, b_in, W_e, b_e, W_h, b_h):
    N, D = x.shape
    E = edge_index.shape[1]
    depth = W_h.shape[0]

    grain = _NC * _NS * _CH * 4
    E_pad = ((E + grain - 1) // grain) * grain
    pad = E_pad - E
    N_pad = ((N + _NS * 8 - 1) // (_NS * 8)) * (_NS * 8)

    src = jnp.concatenate([edge_index[0].astype(jnp.int32),
                           jnp.zeros((pad,), jnp.int32)])
    dst = jnp.concatenate([edge_index[1].astype(jnp.int32),
                           jnp.zeros((pad,), jnp.int32)])
    ea_pad = jnp.pad(edge_attr, ((0, pad), (0, 0)))

    h = _tc_proj(x, W_in, b_in, blk=2000)
    e = _tc_edge_proj(ea_pad, W_e, b_e, n_real=E, blk=2048)

    chunks_per_s = E_pad // (_NS * _CH)
    c0 = min(chunks_per_s - 4, ((chunks_per_s * 7 + 9) // 10 + 3) // 4 * 4)
    sc_layer = _make_sc_layer(N, D, E_pad, N_pad, c0, chunks_per_s - c0)
    for i in range(depth):
        agg = sc_layer(h, e, src, dst)
        h = _tc_update(h, agg[:N], agg[N_pad:N_pad + N], W_h[i], b_h[i],
                       blk=2000)
    return h
